# Initial kernel scaffold; baseline (speedup 1.0000x reference)
import jax
import jax.numpy as jnp
from jax.experimental import pallas as pl


def kernel(x, A_memory, N_memory, W_mu, b_mu, W_var, b_var, epsilon):
    raise NotImplementedError("write your pallas kernel here")


# speedup vs baseline: 1.7037x; 1.7037x over previous
import jax
import jax.numpy as jnp
from jax.experimental import pallas as pl
from jax.experimental.pallas import tpu as pltpu

B = 64
MID = 32
T = 256
D = 512
K_SLOT = D // 16 + 1
K_T = T // 16 + 1
_INV_SQRT_D = 1.0 / (float(D) ** 0.5)
_NEG = -1e30


def _mm(a, b):
    return jax.lax.dot_general(
        a, b, (((1,), (0,)), ((), ())),
        preferred_element_type=jnp.float32,
        precision=jax.lax.Precision.HIGHEST,
    )


def _topk_slot_sum(att):
    def body(_, cur):
        m = jnp.max(cur, axis=-1, keepdims=True)
        return jnp.where(cur == m, _NEG, cur)
    cur = jax.lax.fori_loop(0, K_SLOT - 1, body, att)
    thr = jnp.max(cur, axis=-1, keepdims=True)
    gt = (att > thr).astype(jnp.float32)
    cnt_gt = jnp.sum(gt, axis=-1, keepdims=True)
    sum_gt = jnp.sum(att * gt, axis=-1, keepdims=True)
    return sum_gt + thr * (K_SLOT - cnt_gt)


def _topk_t_mask(col, tril):
    def body(_, cur):
        m = jnp.max(cur, axis=0, keepdims=True)
        return jnp.where(cur == m, _NEG, cur)
    cur = jax.lax.fori_loop(0, K_T - 1, body, col)
    thr = jnp.max(cur, axis=0, keepdims=True)
    gt = col > thr
    cnt_gt = jnp.sum(gt.astype(jnp.float32), axis=0, keepdims=True)
    eq = col == thr
    ceq = _mm(tril, eq.astype(jnp.float32))
    mask = jnp.logical_or(
        gt, jnp.logical_and(eq, ceq <= (K_T - cnt_gt)))
    return mask.astype(jnp.float32)


def _masked_mean(mask_col, mat):
    return jnp.sum(mask_col * mat, axis=0, keepdims=True) * (1.0 / K_T)


def _norm_last(v):
    return jnp.sqrt(jnp.sum(v * v, axis=-1, keepdims=True))


def _body(xn_ref, xa_ref, eps_ref, am_ref, amt_ref, nm_ref, nmt_ref,
          wmut_ref, wvart_ref, bmu_ref, bvar_ref,
          zn_ref, za_ref, aatt_ref, natt_ref, anatt_ref, naatt_ref,
          tml_ref, kl_ref, dist_ref):
    i = pl.program_id(0)

    nx = xn_ref[0]
    ax = xa_ref[0]
    eps = eps_ref[0]
    amt = amt_ref[...]
    nmt = nmt_ref[...]

    att_A = jax.nn.sigmoid(_mm(ax, amt) * _INV_SQRT_D)
    att_NA = jax.nn.sigmoid(_mm(ax, nmt) * _INV_SQRT_D)
    att_AN = jax.nn.sigmoid(_mm(nx, amt) * _INV_SQRT_D)
    att_N = jax.nn.sigmoid(_mm(nx, nmt) * _INV_SQRT_D)

    scale = 1.0 / K_SLOT
    tA = _topk_slot_sum(att_A) * scale
    tNA = _topk_slot_sum(att_NA) * scale
    tAN = _topk_slot_sum(att_AN) * scale
    tN = _topk_slot_sum(att_N) * scale
    aatt_ref[0] = tA
    naatt_ref[0] = tNA
    anatt_ref[0] = tAN
    natt_ref[0] = tN

    am = am_ref[...]
    nm = nm_ref[...]
    A_aug = _mm(att_A, am)
    N_Aaug = _mm(att_NA, nm)
    A_Naug = _mm(att_AN, am)
    N_aug = _mm(att_N, nm)

    ii = jax.lax.broadcasted_iota(jnp.int32, (T, T), 0)
    jj = jax.lax.broadcasted_iota(jnp.int32, (T, T), 1)
    tril = (jj <= ii).astype(jnp.float32)
    mask_A = _topk_t_mask(tA, tril)
    mask_N = _topk_t_mask(tN, tril)
    mask_P = _topk_t_mask(tNA, tril)

    negative_ax = _masked_mean(mask_A, ax)
    anchor_nx = _masked_mean(mask_N, nx)
    positive_nx = _masked_mean(mask_P, ax)

    a = anchor_nx / _norm_last(anchor_nx)
    p = positive_nx / _norm_last(positive_nx)
    n = negative_ax / _norm_last(negative_ax)
    dp = _norm_last(a - p + 1e-6)
    dn = _norm_last(a - n + 1e-6)
    hinge = jnp.maximum(dp - dn + 1.0, 0.0)

    wmut = wmut_ref[...]
    wvart = wvart_ref[...]
    bmu = bmu_ref[...]
    bvar = bvar_ref[...]
    N_aug_mu = _mm(N_aug, wmut) + bmu
    N_aug_var = _mm(N_aug, wvart) + bvar
    std = jnp.sqrt(jnp.exp(N_aug_var))
    N_aug_new = N_aug_mu + eps * std

    A_aug_new = _mm(A_aug, wmut) + bmu
    anchor_nx_new = _masked_mean(mask_N, N_aug_new)
    negative_ax_new = _masked_mean(mask_A, A_aug_new)

    kl_i = jnp.sum(1.0 + N_aug_var - N_aug_mu * N_aug_mu
                   - jnp.exp(N_aug_var))

    A_Naug2 = _mm(A_Naug, wmut) + bmu
    N_Aaug2 = _mm(N_Aaug, wmut) + bmu

    dist_i = jnp.maximum(
        100.0 - _norm_last(negative_ax_new) + _norm_last(anchor_nx_new), 0.0)

    zn_ref[0] = N_aug_new + A_Naug2
    za_ref[0] = A_aug_new + N_Aaug2

    @pl.when(i == 0)
    def _init():
        tml_ref[...] = jnp.zeros_like(tml_ref)
        kl_ref[...] = jnp.zeros_like(kl_ref)
        dist_ref[...] = jnp.zeros_like(dist_ref)

    tml_ref[...] += hinge * (1.0 / MID)
    kl_ref[...] += jnp.full((1, 1), kl_i * (-0.5 / (MID * D)), jnp.float32)
    dist_ref[...] += dist_i * (1.0 / MID)


def kernel(x, A_memory, N_memory, W_mu, b_mu, W_var, b_var, epsilon):
    xn = x[:MID]
    xa = x[MID:]

    sample_spec = pl.BlockSpec((1, T, D), lambda i: (i, 0, 0))
    weight_spec = pl.BlockSpec((D, D), lambda i: (0, 0))
    bias_spec = pl.BlockSpec((1, D), lambda i: (0, 0))
    att_spec = pl.BlockSpec((1, T, 1), lambda i: (i, 0, 0))
    scalar_spec = pl.BlockSpec((1, 1), lambda i: (0, 0))

    out_shapes = (
        jax.ShapeDtypeStruct((MID, T, D), jnp.float32),
        jax.ShapeDtypeStruct((MID, T, D), jnp.float32),
        jax.ShapeDtypeStruct((MID, T, 1), jnp.float32),
        jax.ShapeDtypeStruct((MID, T, 1), jnp.float32),
        jax.ShapeDtypeStruct((MID, T, 1), jnp.float32),
        jax.ShapeDtypeStruct((MID, T, 1), jnp.float32),
        jax.ShapeDtypeStruct((1, 1), jnp.float32),
        jax.ShapeDtypeStruct((1, 1), jnp.float32),
        jax.ShapeDtypeStruct((1, 1), jnp.float32),
    )

    zn, za, aatt, natt, anatt, naatt, tml, kl, dist = pl.pallas_call(
        _body,
        grid=(MID,),
        in_specs=[
            sample_spec, sample_spec, sample_spec,
            weight_spec, weight_spec, weight_spec, weight_spec,
            weight_spec, weight_spec,
            bias_spec, bias_spec,
        ],
        out_specs=(
            sample_spec, sample_spec,
            att_spec, att_spec, att_spec, att_spec,
            scalar_spec, scalar_spec, scalar_spec,
        ),
        out_shape=out_shapes,
        compiler_params=pltpu.CompilerParams(
            dimension_semantics=("arbitrary",),
        ),
    )(
        xn, xa, epsilon,
        A_memory, A_memory.T, N_memory, N_memory.T,
        W_mu.T, W_var.T,
        b_mu.reshape(1, D), b_var.reshape(1, D),
    )

    F_M = jnp.concatenate([x, jnp.concatenate([zn, za], axis=0)], axis=-1)
    return (
        F_M,
        tml.reshape(()),
        kl.reshape(()),
        dist.reshape(()),
        aatt.reshape(MID, T),
        natt.reshape(MID, T),
        anatt.reshape(MID, T),
        naatt.reshape(MID, T),
    )


# speedup vs baseline: 1.7949x; 1.0535x over previous
import jax
import jax.numpy as jnp
from jax.experimental import pallas as pl
from jax.experimental.pallas import tpu as pltpu

B = 64
MID = 32
T = 256
D = 512
K_SLOT = D // 16 + 1
K_T = T // 16 + 1
_INV_SQRT_D = 1.0 / (float(D) ** 0.5)
_NEG = -1e30


def _mm(a, b):
    return jax.lax.dot_general(
        a, b, (((1,), (0,)), ((), ())),
        preferred_element_type=jnp.float32,
        precision=jax.lax.Precision.HIGHEST,
    )


def _topk_slot_sum(att):
    def body(_, cur):
        m = jnp.max(cur, axis=-1, keepdims=True)
        return jnp.where(cur == m, _NEG, cur)
    cur = jax.lax.fori_loop(0, K_SLOT - 1, body, att)
    thr = jnp.max(cur, axis=-1, keepdims=True)
    gt = (att > thr).astype(jnp.float32)
    cnt_gt = jnp.sum(gt, axis=-1, keepdims=True)
    sum_gt = jnp.sum(att * gt, axis=-1, keepdims=True)
    return sum_gt + thr * (K_SLOT - cnt_gt)


def _topk_t_mask(col, tril):
    def body(_, cur):
        m = jnp.max(cur, axis=0, keepdims=True)
        return jnp.where(cur == m, _NEG, cur)
    cur = jax.lax.fori_loop(0, K_T - 1, body, col)
    thr = jnp.max(cur, axis=0, keepdims=True)
    gt = col > thr
    cnt_gt = jnp.sum(gt.astype(jnp.float32), axis=0, keepdims=True)
    eq = col == thr
    ceq = _mm(tril, eq.astype(jnp.float32))
    mask = jnp.logical_or(
        gt, jnp.logical_and(eq, ceq <= (K_T - cnt_gt)))
    return mask.astype(jnp.float32)


def _masked_mean(mask_col, mat):
    return jnp.sum(mask_col * mat, axis=0, keepdims=True) * (1.0 / K_T)


def _norm_last(v):
    return jnp.sqrt(jnp.sum(v * v, axis=-1, keepdims=True))


def _body(x_ref, eps_ref, am_ref, amt_ref, nm_ref, nmt_ref,
          wmut_ref, wvart_ref, bmu_ref, bvar_ref,
          fm_ref, aatt_ref, natt_ref, anatt_ref, naatt_ref,
          tml_ref, kl_ref, dist_ref):
    i = pl.program_id(0)

    nx = x_ref[0, 0]
    ax = x_ref[1, 0]
    eps = eps_ref[0]
    amt = amt_ref[...]
    nmt = nmt_ref[...]

    att_A = jax.nn.sigmoid(_mm(ax, amt) * _INV_SQRT_D)
    att_NA = jax.nn.sigmoid(_mm(ax, nmt) * _INV_SQRT_D)
    att_AN = jax.nn.sigmoid(_mm(nx, amt) * _INV_SQRT_D)
    att_N = jax.nn.sigmoid(_mm(nx, nmt) * _INV_SQRT_D)

    scale = 1.0 / K_SLOT
    tA = _topk_slot_sum(att_A) * scale
    tNA = _topk_slot_sum(att_NA) * scale
    tAN = _topk_slot_sum(att_AN) * scale
    tN = _topk_slot_sum(att_N) * scale
    aatt_ref[0] = tA
    naatt_ref[0] = tNA
    anatt_ref[0] = tAN
    natt_ref[0] = tN

    am = am_ref[...]
    nm = nm_ref[...]
    A_aug = _mm(att_A, am)
    N_Aaug = _mm(att_NA, nm)
    A_Naug = _mm(att_AN, am)
    N_aug = _mm(att_N, nm)

    ii = jax.lax.broadcasted_iota(jnp.int32, (T, T), 0)
    jj = jax.lax.broadcasted_iota(jnp.int32, (T, T), 1)
    tril = (jj <= ii).astype(jnp.float32)
    mask_A = _topk_t_mask(tA, tril)
    mask_N = _topk_t_mask(tN, tril)
    mask_P = _topk_t_mask(tNA, tril)

    negative_ax = _masked_mean(mask_A, ax)
    anchor_nx = _masked_mean(mask_N, nx)
    positive_nx = _masked_mean(mask_P, ax)

    a = anchor_nx / _norm_last(anchor_nx)
    p = positive_nx / _norm_last(positive_nx)
    n = negative_ax / _norm_last(negative_ax)
    dp = _norm_last(a - p + 1e-6)
    dn = _norm_last(a - n + 1e-6)
    hinge = jnp.maximum(dp - dn + 1.0, 0.0)

    wmut = wmut_ref[...]
    wvart = wvart_ref[...]
    bmu = bmu_ref[...]
    bvar = bvar_ref[...]
    N_aug_mu = _mm(N_aug, wmut) + bmu
    N_aug_var = _mm(N_aug, wvart) + bvar
    std = jnp.sqrt(jnp.exp(N_aug_var))
    N_aug_new = N_aug_mu + eps * std

    A_aug_new = _mm(A_aug, wmut) + bmu
    anchor_nx_new = _masked_mean(mask_N, N_aug_new)
    negative_ax_new = _masked_mean(mask_A, A_aug_new)

    kl_i = jnp.sum(1.0 + N_aug_var - N_aug_mu * N_aug_mu
                   - jnp.exp(N_aug_var))

    A_Naug2 = _mm(A_Naug, wmut) + bmu
    N_Aaug2 = _mm(N_Aaug, wmut) + bmu

    dist_i = jnp.maximum(
        100.0 - _norm_last(negative_ax_new) + _norm_last(anchor_nx_new), 0.0)

    fm_ref[0, 0, :, :D] = nx
    fm_ref[0, 0, :, D:] = N_aug_new + A_Naug2
    fm_ref[1, 0, :, :D] = ax
    fm_ref[1, 0, :, D:] = A_aug_new + N_Aaug2

    @pl.when(i == 0)
    def _init():
        tml_ref[...] = jnp.zeros_like(tml_ref)
        kl_ref[...] = jnp.zeros_like(kl_ref)
        dist_ref[...] = jnp.zeros_like(dist_ref)

    tml_ref[...] += hinge * (1.0 / MID)
    kl_ref[...] += jnp.full((1, 1), kl_i * (-0.5 / (MID * D)), jnp.float32)
    dist_ref[...] += dist_i * (1.0 / MID)


def kernel(x, A_memory, N_memory, W_mu, b_mu, W_var, b_var, epsilon):
    x4 = x.reshape(2, MID, T, D)

    pair_spec = pl.BlockSpec((2, 1, T, D), lambda i: (0, i, 0, 0))
    sample_spec = pl.BlockSpec((1, T, D), lambda i: (i, 0, 0))
    weight_spec = pl.BlockSpec((D, D), lambda i: (0, 0))
    bias_spec = pl.BlockSpec((1, D), lambda i: (0, 0))
    fm_spec = pl.BlockSpec((2, 1, T, 2 * D), lambda i: (0, i, 0, 0))
    att_spec = pl.BlockSpec((1, T, 1), lambda i: (i, 0, 0))
    scalar_spec = pl.BlockSpec((1, 1), lambda i: (0, 0))

    out_shapes = (
        jax.ShapeDtypeStruct((2, MID, T, 2 * D), jnp.float32),
        jax.ShapeDtypeStruct((MID, T, 1), jnp.float32),
        jax.ShapeDtypeStruct((MID, T, 1), jnp.float32),
        jax.ShapeDtypeStruct((MID, T, 1), jnp.float32),
        jax.ShapeDtypeStruct((MID, T, 1), jnp.float32),
        jax.ShapeDtypeStruct((1, 1), jnp.float32),
        jax.ShapeDtypeStruct((1, 1), jnp.float32),
        jax.ShapeDtypeStruct((1, 1), jnp.float32),
    )

    fm, aatt, natt, anatt, naatt, tml, kl, dist = pl.pallas_call(
        _body,
        grid=(MID,),
        in_specs=[
            pair_spec, sample_spec,
            weight_spec, weight_spec, weight_spec, weight_spec,
            weight_spec, weight_spec,
            bias_spec, bias_spec,
        ],
        out_specs=(
            fm_spec,
            att_spec, att_spec, att_spec, att_spec,
            scalar_spec, scalar_spec, scalar_spec,
        ),
        out_shape=out_shapes,
        compiler_params=pltpu.CompilerParams(
            dimension_semantics=("arbitrary",),
        ),
    )(
        x4, epsilon,
        A_memory, A_memory.T, N_memory, N_memory.T,
        W_mu.T, W_var.T,
        b_mu.reshape(1, D), b_var.reshape(1, D),
    )

    return (
        fm.reshape(B, T, 2 * D),
        tml.reshape(()),
        kl.reshape(()),
        dist.reshape(()),
        aatt.reshape(MID, T),
        natt.reshape(MID, T),
        anatt.reshape(MID, T),
        naatt.reshape(MID, T),
    )


# speedup vs baseline: 2.4444x; 1.3619x over previous
import jax
import jax.numpy as jnp
from jax.experimental import pallas as pl
from jax.experimental.pallas import tpu as pltpu

B = 64
MID = 32
T = 256
D = 512
K_SLOT = D // 16 + 1
K_T = T // 16 + 1
_INV_SQRT_D = 1.0 / (float(D) ** 0.5)
_NEG = -1e30


def _mm(a, b):
    return jax.lax.dot_general(
        a, b, (((1,), (0,)), ((), ())),
        preferred_element_type=jnp.float32,
        precision=jax.lax.Precision.HIGHEST,
    )


def _mm_fast(a, b):
    return jax.lax.dot_general(
        a.astype(jnp.bfloat16), b.astype(jnp.bfloat16),
        (((1,), (0,)), ((), ())),
        preferred_element_type=jnp.float32,
    )


def _topk_slot_sum(att):
    def body(_, cur):
        m = jnp.max(cur, axis=-1, keepdims=True)
        return jnp.where(cur == m, _NEG, cur)
    cur = jax.lax.fori_loop(0, K_SLOT - 1, body, att)
    thr = jnp.max(cur, axis=-1, keepdims=True)
    gt = (att > thr).astype(jnp.float32)
    cnt_gt = jnp.sum(gt, axis=-1, keepdims=True)
    sum_gt = jnp.sum(att * gt, axis=-1, keepdims=True)
    return sum_gt + thr * (K_SLOT - cnt_gt)


def _topk_t_mask(col, tril):
    def body(_, cur):
        m = jnp.max(cur, axis=0, keepdims=True)
        return jnp.where(cur == m, _NEG, cur)
    cur = jax.lax.fori_loop(0, K_T - 1, body, col)
    thr = jnp.max(cur, axis=0, keepdims=True)
    gt = col > thr
    cnt_gt = jnp.sum(gt.astype(jnp.float32), axis=0, keepdims=True)
    eq = col == thr
    ceq = _mm(tril, eq.astype(jnp.float32))
    mask = jnp.logical_or(
        gt, jnp.logical_and(eq, ceq <= (K_T - cnt_gt)))
    return mask.astype(jnp.float32)


def _masked_mean(mask_col, mat):
    return jnp.sum(mask_col * mat, axis=0, keepdims=True) * (1.0 / K_T)


def _norm_last(v):
    return jnp.sqrt(jnp.sum(v * v, axis=-1, keepdims=True))


def _body(x_ref, eps_ref, am_ref, nm_ref, amnt_ref,
          wmut_ref, wvart_ref, bmu_ref, bvar_ref,
          fm_ref, aatt_ref, natt_ref, anatt_ref, naatt_ref,
          tml_ref, kl_ref, dist_ref):
    i = pl.program_id(0)

    S = x_ref[...].reshape(2 * T, D)
    nx = S[:T]
    ax = S[T:]
    eps = eps_ref[0]

    logits = _mm(S, amnt_ref[...]) * _INV_SQRT_D
    att_am = jax.nn.sigmoid(logits[:, :D])
    att_nm = jax.nn.sigmoid(logits[:, D:])

    scale = 1.0 / K_SLOT
    t_am = _topk_slot_sum(att_am) * scale
    t_nm = _topk_slot_sum(att_nm) * scale
    tAN, tA = t_am[:T], t_am[T:]
    tN, tNA = t_nm[:T], t_nm[T:]
    aatt_ref[0] = tA
    naatt_ref[0] = tNA
    anatt_ref[0] = tAN
    natt_ref[0] = tN

    aug_am = _mm_fast(att_am, am_ref[...])
    aug_nm = _mm_fast(att_nm, nm_ref[...])

    ii = jax.lax.broadcasted_iota(jnp.int32, (T, T), 0)
    jj = jax.lax.broadcasted_iota(jnp.int32, (T, T), 1)
    tril = (jj <= ii).astype(jnp.float32)
    mask_A = _topk_t_mask(tA, tril)
    mask_N = _topk_t_mask(tN, tril)
    mask_P = _topk_t_mask(tNA, tril)

    negative_ax = _masked_mean(mask_A, ax)
    anchor_nx = _masked_mean(mask_N, nx)
    positive_nx = _masked_mean(mask_P, ax)

    a = anchor_nx / _norm_last(anchor_nx)
    p = positive_nx / _norm_last(positive_nx)
    n = negative_ax / _norm_last(negative_ax)
    dp = _norm_last(a - p + 1e-6)
    dn = _norm_last(a - n + 1e-6)
    hinge = jnp.maximum(dp - dn + 1.0, 0.0)

    bmu = bmu_ref[...]
    mu_am = _mm_fast(aug_am, wmut_ref[...]) + bmu
    mu_nm = _mm_fast(aug_nm, wmut_ref[...]) + bmu
    A_Naug2, A_aug_new = mu_am[:T], mu_am[T:]
    N_aug_mu, N_Aaug2 = mu_nm[:T], mu_nm[T:]
    N_aug_var = _mm_fast(aug_nm[:T], wvart_ref[...]) + bvar_ref[...]

    std = jnp.sqrt(jnp.exp(N_aug_var))
    N_aug_new = N_aug_mu + eps * std

    anchor_nx_new = _masked_mean(mask_N, N_aug_new)
    negative_ax_new = _masked_mean(mask_A, A_aug_new)

    kl_i = jnp.sum(1.0 + N_aug_var - N_aug_mu * N_aug_mu
                   - jnp.exp(N_aug_var))

    dist_i = jnp.maximum(
        100.0 - _norm_last(negative_ax_new) + _norm_last(anchor_nx_new), 0.0)

    fm_ref[0, 0, :, :D] = nx
    fm_ref[0, 0, :, D:] = N_aug_new + A_Naug2
    fm_ref[1, 0, :, :D] = ax
    fm_ref[1, 0, :, D:] = A_aug_new + N_Aaug2

    @pl.when(i == 0)
    def _init():
        tml_ref[...] = jnp.zeros_like(tml_ref)
        kl_ref[...] = jnp.zeros_like(kl_ref)
        dist_ref[...] = jnp.zeros_like(dist_ref)

    tml_ref[...] += hinge * (1.0 / MID)
    kl_ref[...] += jnp.full((1, 1), kl_i * (-0.5 / (MID * D)), jnp.float32)
    dist_ref[...] += dist_i * (1.0 / MID)


def kernel(x, A_memory, N_memory, W_mu, b_mu, W_var, b_var, epsilon):
    x4 = x.reshape(2, MID, T, D)
    amnt = jnp.concatenate([A_memory.T, N_memory.T], axis=1)

    pair_spec = pl.BlockSpec((2, 1, T, D), lambda i: (0, i, 0, 0))
    sample_spec = pl.BlockSpec((1, T, D), lambda i: (i, 0, 0))
    weight_spec = pl.BlockSpec((D, D), lambda i: (0, 0))
    wide_spec = pl.BlockSpec((D, 2 * D), lambda i: (0, 0))
    bias_spec = pl.BlockSpec((1, D), lambda i: (0, 0))
    fm_spec = pl.BlockSpec((2, 1, T, 2 * D), lambda i: (0, i, 0, 0))
    att_spec = pl.BlockSpec((1, T, 1), lambda i: (i, 0, 0))
    scalar_spec = pl.BlockSpec((1, 1), lambda i: (0, 0))

    out_shapes = (
        jax.ShapeDtypeStruct((2, MID, T, 2 * D), jnp.float32),
        jax.ShapeDtypeStruct((MID, T, 1), jnp.float32),
        jax.ShapeDtypeStruct((MID, T, 1), jnp.float32),
        jax.ShapeDtypeStruct((MID, T, 1), jnp.float32),
        jax.ShapeDtypeStruct((MID, T, 1), jnp.float32),
        jax.ShapeDtypeStruct((1, 1), jnp.float32),
        jax.ShapeDtypeStruct((1, 1), jnp.float32),
        jax.ShapeDtypeStruct((1, 1), jnp.float32),
    )

    fm, aatt, natt, anatt, naatt, tml, kl, dist = pl.pallas_call(
        _body,
        grid=(MID,),
        in_specs=[
            pair_spec, sample_spec,
            weight_spec, weight_spec, wide_spec,
            weight_spec, weight_spec,
            bias_spec, bias_spec,
        ],
        out_specs=(
            fm_spec,
            att_spec, att_spec, att_spec, att_spec,
            scalar_spec, scalar_spec, scalar_spec,
        ),
        out_shape=out_shapes,
        compiler_params=pltpu.CompilerParams(
            dimension_semantics=("arbitrary",),
        ),
    )(
        x4, epsilon,
        A_memory, N_memory, amnt,
        W_mu.T, W_var.T,
        b_mu.reshape(1, D), b_var.reshape(1, D),
    )

    return (
        fm.reshape(B, T, 2 * D),
        tml.reshape(()),
        kl.reshape(()),
        dist.reshape(()),
        aatt.reshape(MID, T),
        natt.reshape(MID, T),
        anatt.reshape(MID, T),
        naatt.reshape(MID, T),
    )
